# baseline (speedup 1.0000x reference)
import functools

import jax
import jax.numpy as jnp
from jax import lax
from jax.experimental import pallas as pl
from jax.experimental.pallas import tpu as pltpu
from jax.experimental.pallas import tpu_sc as plsc

N = 10000
E = 320000
D = 128
NPAD = 10240
CH = 128
NCHUNK = E // CH
NW = 32
CPW = 80
ROWS_PER_TILE = NPAD // 16
TAU = 2.0
V_TH = 1.0
EPS = 1e-5

_mesh = plsc.VectorSubcoreMesh(
    core_axis_name="c", subcore_axis_name="s", num_cores=2, num_subcores=16)


def _zero_vmem_2d(ref, nrows):
    z = jnp.zeros((16,), jnp.float32)

    def body(i, _):
        for m in range(8):
            ref[i, pl.ds(m * 16, 16)] = z
        return 0

    lax.fori_loop(0, nrows, body, 0)


def _zero_vmem_1d(ref, n):
    z = jnp.zeros((16,), jnp.float32)

    def body(i, _):
        ref[pl.ds(i * 16, 16)] = z
        return 0

    lax.fori_loop(0, n // 16, body, 0)


@functools.partial(
    pl.kernel,
    out_type=jax.ShapeDtypeStruct((2, NPAD), jnp.float32),
    mesh=_mesh,
    scratch_types=[
        pltpu.VMEM((2, CH), jnp.int32),
        pltpu.VMEM((CH,), jnp.float32),
        pltpu.VMEM((ROWS_PER_TILE,), jnp.float32),
        pltpu.VMEM_SHARED((NPAD,), jnp.float32),
        pltpu.SemaphoreType.DMA,
        pltpu.SemaphoreType.DMA,
    ],
)
def _deg_kernel(col_hbm, out_hbm, cbuf, ones_v, zbuf, hist_sh, isem0, isem1):
    c = lax.axis_index("c")
    s = lax.axis_index("s")
    wid = s * 2 + c
    isem = (isem0, isem1)

    _zero_vmem_1d(zbuf, ROWS_PER_TILE)
    one = jnp.ones((16,), jnp.float32)
    for m in range(CH // 16):
        ones_v[pl.ds(m * 16, 16)] = one
    pltpu.sync_copy(zbuf, hist_sh.at[pl.ds(s * ROWS_PER_TILE, ROWS_PER_TILE)])
    plsc.subcore_barrier()

    def chunk(k):
        return jnp.minimum(wid + k * NW, NCHUNK)

    pltpu.async_copy(col_hbm.at[chunk(0)], cbuf.at[0], isem[0])
    pltpu.async_copy(col_hbm.at[chunk(1)], cbuf.at[1], isem[1])

    def outer(i, _):
        for b in range(2):
            k = i * 2 + b
            pltpu.make_async_copy(col_hbm.at[0], cbuf.at[b], isem[b]).wait()
            pltpu.sync_copy(ones_v, hist_sh.at[cbuf.at[b]], add=True)
            pltpu.async_copy(col_hbm.at[chunk(k + 2)], cbuf.at[b], isem[b])
        return 0

    lax.fori_loop(0, CPW // 2, outer, 0)
    pltpu.make_async_copy(col_hbm.at[0], cbuf.at[0], isem[0]).wait()
    pltpu.make_async_copy(col_hbm.at[0], cbuf.at[1], isem[1]).wait()

    plsc.subcore_barrier()
    pltpu.sync_copy(hist_sh.at[pl.ds(s * ROWS_PER_TILE, ROWS_PER_TILE)],
                    out_hbm.at[c, pl.ds(s * ROWS_PER_TILE, ROWS_PER_TILE)])


def _proj_body(x_ref, w_ref, degc_ref, g_ref, sfac_ref, msk_ref):
    deg = degc_ref[:, 0:1] + degc_ref[:, 1:2]
    dinv = jnp.where(deg > 0, 1.0 / jnp.sqrt(jnp.maximum(deg, 1e-12)), 0.0)
    sfac_ref[...] = dinv / jnp.maximum(deg, 1.0)
    msk_ref[...] = (deg > 0).astype(jnp.float32)
    h = lax.dot_general(x_ref[...], w_ref[...], (((1,), (1,)), ((), ())),
                        preferred_element_type=jnp.float32)
    g_ref[...] = h * dinv[:N, :]


_proj = pl.pallas_call(
    _proj_body,
    out_shape=(
        jax.ShapeDtypeStruct((N, D), jnp.float32),
        jax.ShapeDtypeStruct((NPAD, 1), jnp.float32),
        jax.ShapeDtypeStruct((NPAD, 1), jnp.float32),
    ),
)


@functools.partial(
    pl.kernel,
    out_type=jax.ShapeDtypeStruct((2, NPAD, D), jnp.float32),
    mesh=_mesh,
    scratch_types=[
        pltpu.VMEM((2, 2, CH), jnp.int32),
        pltpu.VMEM((2, CH, D), jnp.float32),
        pltpu.VMEM((64, D), jnp.float32),
        pltpu.VMEM_SHARED((NPAD, D), jnp.float32),
        pltpu.SemaphoreType.DMA,
        pltpu.SemaphoreType.DMA,
        pltpu.SemaphoreType.DMA,
        pltpu.SemaphoreType.DMA,
    ],
)
def _scatter_kernel(g_hbm, eidx_hbm, out_hbm,
                    ibuf, rows, zbuf, agg_sh, gsem0, gsem1, isem0, isem1):
    c = lax.axis_index("c")
    s = lax.axis_index("s")
    wid = s * 2 + c
    gsem = (gsem0, gsem1)
    isem = (isem0, isem1)

    _zero_vmem_2d(zbuf, 64)
    for k in range(ROWS_PER_TILE // 64):
        pltpu.sync_copy(zbuf, agg_sh.at[pl.ds(s * ROWS_PER_TILE + k * 64, 64)])
    plsc.subcore_barrier()

    def chunk(k):
        return jnp.minimum(wid + k * NW, NCHUNK)

    pltpu.sync_copy(eidx_hbm.at[chunk(0)], ibuf.at[0])
    pltpu.async_copy(g_hbm.at[ibuf.at[0, 0]], rows.at[0], gsem[0])
    pltpu.async_copy(eidx_hbm.at[chunk(1)], ibuf.at[1], isem[1])

    def outer(i, _):
        for b in range(2):
            k = i * 2 + b
            bn = (b + 1) % 2
            pltpu.make_async_copy(eidx_hbm.at[0], ibuf.at[bn], isem[bn]).wait()
            pltpu.async_copy(g_hbm.at[ibuf.at[bn, 0]], rows.at[bn], gsem[bn])
            pltpu.make_async_copy(
                g_hbm.at[ibuf.at[b, 0]], rows.at[b], gsem[b]).wait()
            pltpu.sync_copy(rows.at[b], agg_sh.at[ibuf.at[b, 1]], add=True)
            pltpu.async_copy(eidx_hbm.at[chunk(k + 2)], ibuf.at[b], isem[b])
        return 0

    lax.fori_loop(0, CPW // 2, outer, 0)
    pltpu.make_async_copy(g_hbm.at[ibuf.at[0, 0]], rows.at[0], gsem[0]).wait()
    pltpu.make_async_copy(eidx_hbm.at[0], ibuf.at[1], isem[1]).wait()
    plsc.subcore_barrier()

    for k in range(ROWS_PER_TILE // 64):
        r0 = s * ROWS_PER_TILE + k * 64
        pltpu.sync_copy(agg_sh.at[pl.ds(r0, 64)],
                        out_hbm.at[c, pl.ds(r0, 64), :])


def _epi_body(aggp_ref, sfac_ref, msk_ref, cb_ref, bnw_ref, bnb_ref, out_ref):
    a = aggp_ref[0, :N, :] + aggp_ref[1, :N, :]
    out = a * sfac_ref[:N, :] + msk_ref[:N, :] * cb_ref[...]
    mean = jnp.mean(out, axis=0, keepdims=True)
    var = jnp.mean((out - mean) * (out - mean), axis=0, keepdims=True)
    y = (out - mean) / jnp.sqrt(var + EPS) * bnw_ref[...] + bnb_ref[...]
    out_ref[...] = (y / TAU >= V_TH).astype(jnp.float32)


_epilogue = pl.pallas_call(
    _epi_body,
    out_shape=jax.ShapeDtypeStruct((N, D), jnp.float32),
)


def kernel(x, edge_index, conv_w, conv_b, lin_res_w, lin_res_b, bn_w, bn_b):
    del lin_res_w, lin_res_b
    ei = edge_index.astype(jnp.int32)
    col = ei[1]
    col_ext = jnp.concatenate(
        [col.reshape(NCHUNK, CH),
         jnp.full((1, CH), N, jnp.int32)], axis=0)
    eidx_ext = jnp.concatenate(
        [ei.reshape(2, NCHUNK, CH).transpose(1, 0, 2),
         jnp.stack([jnp.zeros((1, CH), jnp.int32),
                    jnp.full((1, CH), N, jnp.int32)], axis=1)], axis=0)

    degp = _deg_kernel(col_ext)
    degc = jnp.transpose(degp)
    g, sfac, msk = _proj(x, conv_w, degc)
    aggp = _scatter_kernel(g, eidx_ext)
    spike = _epilogue(aggp, sfac, msk,
                      conv_b.reshape(1, D),
                      bn_w.reshape(1, D), bn_b.reshape(1, D))
    return spike


# speedup vs baseline: 1.1960x; 1.1960x over previous
import functools

import jax
import jax.numpy as jnp
from jax import lax
from jax.experimental import pallas as pl
from jax.experimental.pallas import tpu as pltpu
from jax.experimental.pallas import tpu_sc as plsc

N = 10000
E = 320000
D = 128
NPAD = 10240
CH = 128
NCHUNK = E // CH
NW = 32
CPW = 80
ROWS_PER_TILE = NPAD // 16
TAU = 2.0
V_TH = 1.0
EPS = 1e-5

_mesh = plsc.VectorSubcoreMesh(
    core_axis_name="c", subcore_axis_name="s", num_cores=2, num_subcores=16)


def _zero_vmem_2d(ref, nrows):
    z = jnp.zeros((16,), jnp.float32)

    def body(i, _):
        for m in range(8):
            ref[i, pl.ds(m * 16, 16)] = z
        return 0

    lax.fori_loop(0, nrows, body, 0)


def _zero_vmem_1d(ref, n):
    z = jnp.zeros((16,), jnp.float32)

    def body(i, _):
        ref[pl.ds(i * 16, 16)] = z
        return 0

    lax.fori_loop(0, n // 16, body, 0)


@functools.partial(
    pl.kernel,
    out_type=jax.ShapeDtypeStruct((2, NPAD), jnp.float32),
    mesh=_mesh,
    scratch_types=[
        pltpu.VMEM((2, CH), jnp.int32),
        pltpu.VMEM((CH,), jnp.float32),
        pltpu.VMEM((ROWS_PER_TILE,), jnp.float32),
        pltpu.VMEM_SHARED((NPAD,), jnp.float32),
        pltpu.SemaphoreType.DMA,
        pltpu.SemaphoreType.DMA,
    ],
)
def _deg_kernel(col_hbm, out_hbm, cbuf, ones_v, zbuf, hist_sh, isem0, isem1):
    c = lax.axis_index("c")
    s = lax.axis_index("s")
    wid = s * 2 + c
    isem = (isem0, isem1)

    _zero_vmem_1d(zbuf, ROWS_PER_TILE)
    one = jnp.ones((16,), jnp.float32)
    for m in range(CH // 16):
        ones_v[pl.ds(m * 16, 16)] = one
    pltpu.sync_copy(zbuf, hist_sh.at[pl.ds(s * ROWS_PER_TILE, ROWS_PER_TILE)])
    plsc.subcore_barrier()

    def chunk(k):
        return jnp.minimum(wid + k * NW, NCHUNK)

    pltpu.async_copy(col_hbm.at[chunk(0)], cbuf.at[0], isem[0])
    pltpu.async_copy(col_hbm.at[chunk(1)], cbuf.at[1], isem[1])

    def outer(i, _):
        for b in range(2):
            k = i * 2 + b
            pltpu.make_async_copy(col_hbm.at[0], cbuf.at[b], isem[b]).wait()
            pltpu.sync_copy(ones_v, hist_sh.at[cbuf.at[b]], add=True)
            pltpu.async_copy(col_hbm.at[chunk(k + 2)], cbuf.at[b], isem[b])
        return 0

    lax.fori_loop(0, CPW // 2, outer, 0)
    pltpu.make_async_copy(col_hbm.at[0], cbuf.at[0], isem[0]).wait()
    pltpu.make_async_copy(col_hbm.at[0], cbuf.at[1], isem[1]).wait()

    plsc.subcore_barrier()
    pltpu.sync_copy(hist_sh.at[pl.ds(s * ROWS_PER_TILE, ROWS_PER_TILE)],
                    out_hbm.at[c, pl.ds(s * ROWS_PER_TILE, ROWS_PER_TILE)])


def _proj_body(x_ref, w_ref, degc_ref, g_ref, sfac_ref, msk_ref):
    deg = degc_ref[:, 0:1] + degc_ref[:, 1:2]
    dinv = jnp.where(deg > 0, 1.0 / jnp.sqrt(jnp.maximum(deg, 1e-12)), 0.0)
    sfac_ref[...] = dinv / jnp.maximum(deg, 1.0)
    msk_ref[...] = (deg > 0).astype(jnp.float32)
    h = lax.dot_general(x_ref[...], w_ref[...], (((1,), (1,)), ((), ())),
                        preferred_element_type=jnp.float32)
    g_ref[...] = h * dinv[:N, :]


_proj = pl.pallas_call(
    _proj_body,
    out_shape=(
        jax.ShapeDtypeStruct((N, D), jnp.float32),
        jax.ShapeDtypeStruct((NPAD, 1), jnp.float32),
        jax.ShapeDtypeStruct((NPAD, 1), jnp.float32),
    ),
)


@functools.partial(
    pl.kernel,
    out_type=jax.ShapeDtypeStruct((2, NPAD, D), jnp.float32),
    mesh=_mesh,
    scratch_types=[
        pltpu.VMEM((3, 2, CH), jnp.int32),
        pltpu.VMEM((CH, D), jnp.float32),
        pltpu.VMEM((64, D), jnp.float32),
        pltpu.VMEM_SHARED((NPAD, D), jnp.float32),
        pltpu.SemaphoreType.DMA,
        pltpu.SemaphoreType.DMA,
        pltpu.SemaphoreType.DMA,
        pltpu.SemaphoreType.DMA,
    ],
)
def _scatter_kernel(g_hbm, eidx_hbm, out_hbm,
                    ibuf, rows, zbuf, agg_sh, gsem, isem0, isem1, isem2):
    c = lax.axis_index("c")
    s = lax.axis_index("s")
    wid = s * 2 + c
    isem = (isem0, isem1, isem2)

    _zero_vmem_2d(zbuf, 64)
    for k in range(ROWS_PER_TILE // 64):
        pltpu.sync_copy(zbuf, agg_sh.at[pl.ds(s * ROWS_PER_TILE + k * 64, 64)])
    plsc.subcore_barrier()

    def chunk(k):
        return jnp.minimum(wid + k * NW, NCHUNK)

    pltpu.async_copy(eidx_hbm.at[chunk(0)], ibuf.at[0], isem[0])
    pltpu.async_copy(eidx_hbm.at[chunk(1)], ibuf.at[1], isem[1])

    def outer(i, _):
        for b in range(3):
            k = i * 3 + b
            b2 = (b + 2) % 3
            pltpu.async_copy(eidx_hbm.at[chunk(k + 2)], ibuf.at[b2], isem[b2])
            pltpu.make_async_copy(eidx_hbm.at[0], ibuf.at[b], isem[b]).wait()
            pltpu.async_copy(g_hbm.at[ibuf.at[b, 0]], rows, gsem).wait()
            pltpu.sync_copy(rows, agg_sh.at[ibuf.at[b, 1]], add=True)
        return 0

    lax.fori_loop(0, CPW // 3, outer, 0)
    for b in range(CPW % 3):
        k = (CPW // 3) * 3 + b
        b2 = (b + 2) % 3
        pltpu.async_copy(eidx_hbm.at[chunk(k + 2)], ibuf.at[b2], isem[b2])
        pltpu.make_async_copy(eidx_hbm.at[0], ibuf.at[b], isem[b]).wait()
        pltpu.async_copy(g_hbm.at[ibuf.at[b, 0]], rows, gsem).wait()
        pltpu.sync_copy(rows, agg_sh.at[ibuf.at[b, 1]], add=True)
    bL = CPW % 3
    pltpu.make_async_copy(eidx_hbm.at[0], ibuf.at[bL], isem[bL]).wait()
    pltpu.make_async_copy(
        eidx_hbm.at[0], ibuf.at[(bL + 1) % 3], isem[(bL + 1) % 3]).wait()
    plsc.subcore_barrier()

    for k in range(ROWS_PER_TILE // 64):
        r0 = s * ROWS_PER_TILE + k * 64
        pltpu.sync_copy(agg_sh.at[pl.ds(r0, 64)],
                        out_hbm.at[c, pl.ds(r0, 64), :])


def _epi_body(aggp_ref, sfac_ref, msk_ref, cb_ref, bnw_ref, bnb_ref, out_ref):
    a = aggp_ref[0, :N, :] + aggp_ref[1, :N, :]
    out = a * sfac_ref[:N, :] + msk_ref[:N, :] * cb_ref[...]
    mean = jnp.mean(out, axis=0, keepdims=True)
    var = jnp.mean((out - mean) * (out - mean), axis=0, keepdims=True)
    y = (out - mean) / jnp.sqrt(var + EPS) * bnw_ref[...] + bnb_ref[...]
    out_ref[...] = (y / TAU >= V_TH).astype(jnp.float32)


_epilogue = pl.pallas_call(
    _epi_body,
    out_shape=jax.ShapeDtypeStruct((N, D), jnp.float32),
)


def kernel(x, edge_index, conv_w, conv_b, lin_res_w, lin_res_b, bn_w, bn_b):
    del lin_res_w, lin_res_b
    ei = edge_index.astype(jnp.int32)
    col = ei[1]
    col_ext = jnp.concatenate(
        [col.reshape(NCHUNK, CH),
         jnp.full((1, CH), N, jnp.int32)], axis=0)
    eidx_ext = jnp.concatenate(
        [ei.reshape(2, NCHUNK, CH).transpose(1, 0, 2),
         jnp.stack([jnp.zeros((1, CH), jnp.int32),
                    jnp.full((1, CH), N, jnp.int32)], axis=1)], axis=0)

    degp = _deg_kernel(col_ext)
    degc = jnp.transpose(degp)
    g, sfac, msk = _proj(x, conv_w, degc)
    aggp = _scatter_kernel(g, eidx_ext)
    spike = _epilogue(aggp, sfac, msk,
                      conv_b.reshape(1, D),
                      bn_w.reshape(1, D), bn_b.reshape(1, D))
    return spike
